# baseline (speedup 1.0000x reference)
import jax
import jax.numpy as jnp
from jax.experimental import pallas as pl

_N = 4096
_D = 512
_G = 2
_ALPHA = 0.1
_BM0 = 512
_BM1 = 1024


def _layer0_body(a_ref, x_ref, w_ref, o_ref, a16_ref):
    r = pl.program_id(1)
    a16 = a_ref[0].astype(jnp.bfloat16)
    a16_ref[0] = a16
    x16 = x_ref[0].astype(jnp.bfloat16)
    t = jnp.dot(a16, x16, preferred_element_type=jnp.float32)
    xr = x_ref[0, pl.ds(r * _BM0, _BM0), :]
    t = (1.0 - _ALPHA) * t + _ALPHA * xr
    h = jnp.dot(t.astype(jnp.bfloat16), w_ref[0].astype(jnp.bfloat16),
                preferred_element_type=jnp.float32)
    o_ref[0] = jnp.maximum(h, 0.0).astype(jnp.bfloat16)


def _layer1_body(a_ref, h_ref, w_ref, o_ref):
    r = pl.program_id(0)
    j = pl.program_id(1)
    a16 = a_ref[0]
    hj = h_ref[j]
    t = jnp.dot(a16, hj, preferred_element_type=jnp.float32)
    hr = h_ref[j, pl.ds(r * _BM1, _BM1), :].astype(jnp.float32)
    t = (1.0 - _ALPHA) * t + _ALPHA * hr
    c = jnp.dot(t.astype(jnp.bfloat16), w_ref[j].astype(jnp.bfloat16),
                preferred_element_type=jnp.float32) * (1.0 / _G)

    @pl.when(j == 0)
    def _():
        o_ref[...] = c

    @pl.when(j > 0)
    def _():
        o_ref[...] += c


def kernel(adj_list, x_list, W_0_0, W_0_1, W_1_0, W_1_1):
    w0 = jnp.stack([W_0_0, W_0_1])
    w1 = jnp.stack([W_1_0, W_1_1])

    h16, a16_all = pl.pallas_call(
        _layer0_body,
        grid=(_G, _N // _BM0),
        in_specs=[
            pl.BlockSpec((1, _BM0, _N), lambda j, r: (j, r, 0)),
            pl.BlockSpec((1, _N, _D), lambda j, r: (j, 0, 0)),
            pl.BlockSpec((1, _D, _D), lambda j, r: (j, 0, 0)),
        ],
        out_specs=[
            pl.BlockSpec((1, _BM0, _D), lambda j, r: (j, r, 0)),
            pl.BlockSpec((1, _BM0, _N), lambda j, r: (j, r, 0)),
        ],
        out_shape=[
            jax.ShapeDtypeStruct((_G, _N, _D), jnp.bfloat16),
            jax.ShapeDtypeStruct((_G, _N, _N), jnp.bfloat16),
        ],
    )(adj_list, x_list, w0)

    return pl.pallas_call(
        _layer1_body,
        grid=(_N // _BM1, _G),
        in_specs=[
            pl.BlockSpec((1, _BM1, _N), lambda r, j: (j, r, 0)),
            pl.BlockSpec((_G, _N, _D), lambda r, j: (0, 0, 0)),
            pl.BlockSpec((_G, _D, _D), lambda r, j: (0, 0, 0)),
        ],
        out_specs=pl.BlockSpec((_BM1, _D), lambda r, j: (r, 0)),
        out_shape=jax.ShapeDtypeStruct((_N, _D), jnp.float32),
    )(a16_all, h16, w1)


# speedup vs baseline: 1.1240x; 1.1240x over previous
import jax
import jax.numpy as jnp
from jax.experimental import pallas as pl

_N = 4096
_D = 512
_G = 2
_BM = 1024
_R = _N // _BM
_ALPHA = 0.1


def _layer0_body(a_ref, x_ref, w_ref, o_ref):
    r = pl.program_id(1)
    a16 = a_ref[0].astype(jnp.bfloat16)
    x16 = x_ref[0].astype(jnp.bfloat16)
    t = jnp.dot(a16, x16, preferred_element_type=jnp.float32)
    xr = x_ref[0, pl.ds(r * _BM, _BM), :]
    t = (1.0 - _ALPHA) * t + _ALPHA * xr
    h = jnp.dot(t.astype(jnp.bfloat16), w_ref[0].astype(jnp.bfloat16),
                preferred_element_type=jnp.float32)
    o_ref[0] = jnp.maximum(h, 0.0).astype(jnp.bfloat16)


def _layer1_body(a_ref, h_ref, w_ref, o_ref):
    r = pl.program_id(0)
    j = pl.program_id(1)
    a16 = a_ref[0].astype(jnp.bfloat16)
    hj = h_ref[j]
    t = jnp.dot(a16, hj, preferred_element_type=jnp.float32)
    hr = h_ref[j, pl.ds(r * _BM, _BM), :].astype(jnp.float32)
    t = (1.0 - _ALPHA) * t + _ALPHA * hr
    c = jnp.dot(t.astype(jnp.bfloat16), w_ref[j].astype(jnp.bfloat16),
                preferred_element_type=jnp.float32) * (1.0 / _G)

    @pl.when(j == 0)
    def _():
        o_ref[...] = c

    @pl.when(j > 0)
    def _():
        o_ref[...] += c


def kernel(adj_list, x_list, W_0_0, W_0_1, W_1_0, W_1_1):
    w0 = jnp.stack([W_0_0, W_0_1])
    w1 = jnp.stack([W_1_0, W_1_1])

    h16 = pl.pallas_call(
        _layer0_body,
        grid=(_G, _R),
        in_specs=[
            pl.BlockSpec((1, _BM, _N), lambda j, r: (j, r, 0)),
            pl.BlockSpec((1, _N, _D), lambda j, r: (j, 0, 0)),
            pl.BlockSpec((1, _D, _D), lambda j, r: (j, 0, 0)),
        ],
        out_specs=pl.BlockSpec((1, _BM, _D), lambda j, r: (j, r, 0)),
        out_shape=jax.ShapeDtypeStruct((_G, _N, _D), jnp.bfloat16),
    )(adj_list, x_list, w0)

    return pl.pallas_call(
        _layer1_body,
        grid=(_R, _G),
        in_specs=[
            pl.BlockSpec((1, _BM, _N), lambda r, j: (j, r, 0)),
            pl.BlockSpec((_G, _N, _D), lambda r, j: (0, 0, 0)),
            pl.BlockSpec((_G, _D, _D), lambda r, j: (0, 0, 0)),
        ],
        out_specs=pl.BlockSpec((_BM, _D), lambda r, j: (r, 0)),
        out_shape=jax.ShapeDtypeStruct((_N, _D), jnp.float32),
    )(adj_list, h16, w1)
